# baseline (speedup 1.0000x reference)
import functools
import math

import jax
import jax.numpy as jnp
from jax import lax
from jax.experimental import pallas as pl
from jax.experimental.pallas import tpu as pltpu
from jax.experimental.pallas import tpu_sc as plsc

_L = 16
_NW = 32
_CHUNK = 128


@functools.lru_cache(maxsize=None)
def _build_lookup(seq: int, batch: int, vocab: int, dim: int):
    assert batch % _CHUNK == 0 and dim % _L == 0
    n_chunks = (seq * batch) // _CHUNK
    cpw = n_chunks // _NW
    cps = batch // _CHUNK
    scale = math.sqrt(dim)
    ngroups = _CHUNK // _L
    ncb = dim // _L

    @functools.partial(
        pl.kernel,
        out_type=jax.ShapeDtypeStruct((seq, dim, batch), jnp.float32),
        mesh=plsc.VectorSubcoreMesh(core_axis_name="c", subcore_axis_name="s"),
        compiler_params=pltpu.CompilerParams(use_tc_tiling_on_sc=True,
                                             needs_layout_passes=False),
        scratch_types=[
            pltpu.VMEM((cpw, _CHUNK), jnp.int32),
            pltpu.VMEM((cpw, _CHUNK), jnp.int32),
            pltpu.VMEM((16, dim), jnp.float32),
            pltpu.VMEM((2, _CHUNK, 2 * dim), jnp.float32),
            pltpu.VMEM((2, dim, _CHUNK), jnp.float32),
            pltpu.VMEM((16, dim), jnp.float32),
            pltpu.VMEM((16,), jnp.int32),
            pltpu.SemaphoreType.DMA,
            pltpu.SemaphoreType.DMA,
            pltpu.SemaphoreType.DMA,
        ],
    )
    def lookup(x_hbm, tc_hbm, pe_hbm, out_hbm,
               idx_v, pidx_v, pe_v, gbuf, obuf, tmp_v, ivec_v,
               ssem, gsem, osem):
        wid = lax.axis_index("s") * 2 + lax.axis_index("c")
        base_c = pl.multiple_of(wid * cpw, 8)
        s0 = base_c // cps
        start8 = pl.multiple_of(
            lax.min((s0 // 8) * 8, jnp.int32(seq - 16)), 8)

        pltpu.make_async_copy(x_hbm.at[pl.ds(base_c, cpw)], idx_v,
                              ssem).start()
        pltpu.make_async_copy(pe_hbm.at[pl.ds(start8, 16)], pe_v,
                              ssem).start()
        pltpu.make_async_copy(x_hbm.at[pl.ds(base_c, cpw)], idx_v,
                              ssem).wait()
        pltpu.make_async_copy(pe_hbm.at[pl.ds(start8, 16)], pe_v,
                              ssem).wait()

        def shift(i, c2):
            for k in range(_CHUNK // _L):
                sl = pl.ds(k * _L, _L)
                pidx_v[i, sl] = lax.shift_right_logical(idx_v[i, sl], 1)
            return c2
        lax.fori_loop(0, cpw, shift, 0)

        def gather(t, slot):
            return pltpu.make_async_copy(
                tc_hbm.at[pidx_v.at[t]], gbuf.at[slot], gsem)

        def put(t, slot):
            c = base_c + t
            return pltpu.make_async_copy(
                obuf.at[slot],
                out_hbm.at[c // cps, :,
                           pl.ds(pl.multiple_of(lax.rem(c, cps) * _CHUNK, 8),
                                 _CHUNK)],
                osem)

        gather(0, 0).start()
        gather(1, 1).start()

        ivec_v[pl.ds(0, _L)] = lax.iota(jnp.int32, _L)

        def step(t, carry):
            iota = ivec_v[pl.ds(0, _L)]
            slot = lax.rem(t, 2)
            gather(t, slot).wait()

            @pl.when(t >= 2)
            def _():
                put(t, slot).wait()

            s_loc = (base_c + t) // cps - start8
            s_vec = jnp.broadcast_to(s_loc, (_L,))
            t_vec = jnp.broadcast_to(t, (_L,))
            g_ref = gbuf.at[slot]
            o_ref = obuf.at[slot]

            def group(g, c2):
                for l in range(_L):
                    row_scalar = g * _L + l
                    par_l = lax.bitwise_and(
                        plsc.load_gather(
                            idx_v, [t_vec, jnp.broadcast_to(row_scalar, (_L,))]),
                        1) * dim
                    row_l = jnp.broadcast_to(row_scalar, (_L,))
                    rot = lax.bitwise_and(iota + (_L - l), _L - 1)
                    for cb in range(ncb):
                        tmp_v[l, pl.ds(cb * _L, _L)] = plsc.load_gather(
                            g_ref, [row_l, par_l + (cb * _L) + rot])

                for cb in range(ncb):
                    for c in range(_L):
                        diag = lax.bitwise_and(iota + c, _L - 1)
                        vals = plsc.load_gather(tmp_v, [iota, cb * _L + diag])
                        pe_c = plsc.load_gather(
                            pe_v, [s_vec,
                                   jnp.broadcast_to(jnp.int32(cb * _L + c),
                                                    (_L,))])
                        o_ref[cb * _L + c, pl.ds(g * _L, _L)] = \
                            vals * scale + pe_c
                return c2
            lax.fori_loop(0, ngroups, group, 0)

            put(t, slot).start()

            @pl.when(t + 2 < cpw)
            def _():
                gather(t + 2, slot).start()
            return carry

        lax.fori_loop(0, cpw, step, 0)
        put(cpw - 2, lax.rem(cpw - 2, 2)).wait()
        put(cpw - 1, lax.rem(cpw - 1, 2)).wait()

    return lookup


def kernel(x, table, pe, pos):
    seq, batch = x.shape
    vocab, dim = table.shape
    tablec = table.reshape(vocab // 2, 2 * dim)
    pe_rows = lax.dynamic_slice_in_dim(pe, pos, seq, axis=0).reshape(seq, dim)
    x2 = x.astype(jnp.int32).reshape((seq * batch) // _CHUNK, _CHUNK)
    out_t = _build_lookup(seq, batch, vocab, dim)(x2, tablec, pe_rows)
    return jnp.transpose(out_t, (0, 2, 1))


# speedup vs baseline: 1.8063x; 1.8063x over previous
import functools
import math

import jax
import jax.numpy as jnp
from jax import lax
from jax.experimental import pallas as pl
from jax.experimental.pallas import tpu as pltpu
from jax.experimental.pallas import tpu_sc as plsc

_L = 16
_NW = 32
_BR = 200
_CHUNK = 128

_PARAMS = pltpu.CompilerParams(use_tc_tiling_on_sc=True,
                               needs_layout_passes=False)


def _mesh():
    return plsc.VectorSubcoreMesh(core_axis_name="c", subcore_axis_name="s")


@functools.lru_cache(maxsize=None)
def _build_repad(vocab: int, dim: int):
    assert vocab % _BR == 0 and _BR % 8 == 0
    nblocks = vocab // _BR

    @functools.partial(
        pl.kernel,
        out_type=jax.ShapeDtypeStruct((vocab, 2 * dim), jnp.float32),
        mesh=_mesh(),
        compiler_params=_PARAMS,
        scratch_types=[
            pltpu.VMEM((2, _BR, dim), jnp.float32),
            pltpu.VMEM((2, _BR, 2 * dim), jnp.float32),
            pltpu.SemaphoreType.DMA,
            pltpu.SemaphoreType.DMA,
        ],
    )
    def repad(table_hbm, out_hbm, buf, wbuf, rsem, wsem):
        wid = lax.axis_index("s") * 2 + lax.axis_index("c")
        nt = (nblocks - wid + _NW - 1) // _NW

        def read(b, slot):
            return pltpu.make_async_copy(
                table_hbm.at[pl.ds(pl.multiple_of(b * _BR, 8), _BR)],
                buf.at[slot], rsem)

        def write(b, slot):
            return pltpu.make_async_copy(
                wbuf.at[slot],
                out_hbm.at[pl.ds(pl.multiple_of(b * _BR, 8), _BR)], wsem)

        read(wid, 0).start()
        read(wid + _NW, 1).start()

        def step(t, carry):
            b = wid + t * _NW
            slot = lax.rem(t, 2)
            read(b, slot).wait()

            @pl.when(t >= 2)
            def _():
                write(b, slot).wait()

            def row(i, c2):
                for k in range(dim // _L):
                    sl = pl.ds(k * _L, _L)
                    wbuf[slot, i, sl] = buf[slot, i, sl]
                return c2
            lax.fori_loop(0, _BR, row, 0, unroll=2)

            write(b, slot).start()

            @pl.when(t + 2 < nt)
            def _():
                read(b + 2 * _NW, slot).start()
            return carry

        lax.fori_loop(0, nt, step, 0)
        write(0, lax.rem(nt - 2, 2)).wait()
        write(0, lax.rem(nt - 1, 2)).wait()

    return repad


@functools.lru_cache(maxsize=None)
def _build_lookup(seq: int, batch: int, vocab: int, dim: int):
    assert batch % _CHUNK == 0 and dim % _L == 0
    n_chunks = (seq * batch) // _CHUNK
    cpw = n_chunks // _NW
    cps = batch // _CHUNK
    scale = math.sqrt(dim)
    nk = dim // _L

    @functools.partial(
        pl.kernel,
        out_type=jax.ShapeDtypeStruct((n_chunks, _CHUNK, dim), jnp.float32),
        mesh=_mesh(),
        compiler_params=_PARAMS,
        scratch_types=[
            pltpu.VMEM((cpw, _CHUNK), jnp.int32),
            pltpu.VMEM((16, dim), jnp.float32),
            pltpu.VMEM((2, _CHUNK, 2 * dim), jnp.float32),
            pltpu.VMEM((2, _CHUNK, dim), jnp.float32),
            pltpu.SemaphoreType.DMA,
            pltpu.SemaphoreType.DMA,
            pltpu.SemaphoreType.DMA,
        ],
    )
    def lookup(x_hbm, tp_hbm, pe_hbm, out_hbm,
               idx_v, pe_v, gbuf, obuf, ssem, gsem, osem):
        wid = lax.axis_index("s") * 2 + lax.axis_index("c")
        base_c = pl.multiple_of(wid * cpw, 8)
        s0 = base_c // cps
        start8 = pl.multiple_of(
            lax.min((s0 // 8) * 8, jnp.int32(seq - 16)), 8)

        pltpu.make_async_copy(x_hbm.at[pl.ds(base_c, cpw)], idx_v,
                              ssem).start()
        pltpu.make_async_copy(pe_hbm.at[pl.ds(start8, 16)], pe_v,
                              ssem).start()
        pltpu.make_async_copy(x_hbm.at[pl.ds(base_c, cpw)], idx_v,
                              ssem).wait()
        pltpu.make_async_copy(pe_hbm.at[pl.ds(start8, 16)], pe_v,
                              ssem).wait()

        def gather(t, slot):
            return pltpu.make_async_copy(
                tp_hbm.at[idx_v.at[t]], gbuf.at[slot], gsem)

        def put(t, slot):
            return pltpu.make_async_copy(
                obuf.at[slot], out_hbm.at[base_c + t], osem)

        gather(0, 0).start()
        gather(1, 1).start()

        def step(t, carry):
            slot = lax.rem(t, 2)
            gather(t, slot).wait()

            @pl.when(t >= 2)
            def _():
                put(t, slot).wait()

            s_loc = (base_c + t) // cps - start8
            pe_regs = [pe_v[s_loc, pl.ds(k * _L, _L)] for k in range(nk)]
            g_ref = gbuf.at[slot]
            o_ref = obuf.at[slot]

            def row(i, c2):
                for k in range(nk):
                    sl = pl.ds(k * _L, _L)
                    o_ref[i, sl] = g_ref[i, sl] * scale + pe_regs[k]
                return c2
            lax.fori_loop(0, _CHUNK, row, 0, unroll=2)

            put(t, slot).start()

            @pl.when(t + 2 < cpw)
            def _():
                gather(t + 2, slot).start()
            return carry

        lax.fori_loop(0, cpw, step, 0)
        put(cpw - 2, lax.rem(cpw - 2, 2)).wait()
        put(cpw - 1, lax.rem(cpw - 1, 2)).wait()

    return lookup


def kernel(x, table, pe, pos):
    seq, batch = x.shape
    vocab, dim = table.shape
    tp = jnp.pad(table, ((0, 0), (0, dim)))
    pe_rows = lax.dynamic_slice_in_dim(pe, pos, seq, axis=0).reshape(seq, dim)
    x2 = x.astype(jnp.int32).reshape((seq * batch) // _CHUNK, _CHUNK)
    out = _build_lookup(seq, batch, vocab, dim)(x2, tp, pe_rows)
    return out.reshape(seq, batch, dim)
